# baseline (speedup 1.0000x reference)
import functools

import jax
import jax.numpy as jnp
from jax import lax
from jax.experimental import pallas as pl
from jax.experimental.pallas import tpu as pltpu
from jax.experimental.pallas import tpu_sc as plsc

_N = 100000
_NUM_TYPES = 100
_NC = 2
_NS = 16
_NW = _NC * _NS
_L = 16
_B_PER_W = 3136
_LAST_BASE = _N - _B_PER_W
_UNROLL = 7
_HALF = _B_PER_W // 2
_HVECS = _HALF // _L

_mesh = plsc.VectorSubcoreMesh(core_axis_name="c", subcore_axis_name="s")


@functools.partial(
    pl.kernel,
    out_type=jax.ShapeDtypeStruct((_N,), jnp.float32),
    mesh=_mesh,
    scratch_types=[
        pltpu.VMEM((_B_PER_W,), jnp.int32),
        pltpu.VMEM((_B_PER_W,), jnp.float32),
        pltpu.VMEM((_B_PER_W,), jnp.float32),
        pltpu.VMEM((_NUM_TYPES,), jnp.float32),
        pltpu.SemaphoreType.DMA,
    ],
    compiler_params=pltpu.CompilerParams(needs_layout_passes=False),
)
def _sc_embed_add(idx_hbm, en_hbm, tab_hbm, out_hbm, idx_v, en_v, out_v, tab_v,
                  sem):
    wid = lax.axis_index("s") * _NC + lax.axis_index("c")
    base = lax.min(wid * _B_PER_W, _LAST_BASE)
    c1 = pltpu.async_copy(tab_hbm, tab_v, sem)
    c2 = pltpu.async_copy(idx_hbm.at[pl.ds(base, _B_PER_W)], idx_v, sem)
    c3 = pltpu.async_copy(en_hbm.at[pl.ds(base, _B_PER_W)], en_v, sem)
    c1.wait()
    c2.wait()
    c3.wait()

    @plsc.parallel_loop(0, _HVECS, 1, unroll=_UNROLL)
    def body_lo(j):
        off = j * _L
        iv = idx_v[pl.ds(off, _L)]
        g = plsc.load_gather(tab_v, [iv])
        out_v[pl.ds(off, _L)] = g + en_v[pl.ds(off, _L)]

    c4 = pltpu.async_copy(out_v.at[pl.ds(0, _HALF)],
                          out_hbm.at[pl.ds(base, _HALF)], sem)

    @plsc.parallel_loop(_HVECS, 2 * _HVECS, 1, unroll=_UNROLL)
    def body_hi(j):
        off = j * _L
        iv = idx_v[pl.ds(off, _L)]
        g = plsc.load_gather(tab_v, [iv])
        out_v[pl.ds(off, _L)] = g + en_v[pl.ds(off, _L)]

    c5 = pltpu.async_copy(out_v.at[pl.ds(_HALF, _HALF)],
                          out_hbm.at[pl.ds(base + _HALF, _HALF)], sem)
    c4.wait()
    c5.wait()


def kernel(atom_type, pos, atomic_energy, emb_weight):
    out = _sc_embed_add(
        atom_type.reshape(-1),
        atomic_energy.reshape(-1),
        emb_weight.reshape(-1),
    )
    return out.reshape(_N, 1).astype(pos.dtype)


# speedup vs baseline: 1.0102x; 1.0102x over previous
import functools

import jax
import jax.numpy as jnp
from jax import lax
from jax.experimental import pallas as pl
from jax.experimental.pallas import tpu as pltpu
from jax.experimental.pallas import tpu_sc as plsc

_N = 100000
_NUM_TYPES = 100
_NC = 2
_NS = 16
_NW = _NC * _NS
_L = 16
_B_PER_W = 3136
_LAST_BASE = _N - _B_PER_W
_UNROLL = 7
_VECS = _B_PER_W // _L

_mesh = plsc.VectorSubcoreMesh(core_axis_name="c", subcore_axis_name="s")


@functools.partial(
    pl.kernel,
    out_type=jax.ShapeDtypeStruct((_N,), jnp.float32),
    mesh=_mesh,
    scratch_types=[
        pltpu.VMEM((_B_PER_W,), jnp.int32),
        pltpu.VMEM((_B_PER_W,), jnp.float32),
        pltpu.VMEM((_B_PER_W,), jnp.float32),
        pltpu.VMEM((_NUM_TYPES,), jnp.float32),
        pltpu.SemaphoreType.DMA,
    ],
    compiler_params=pltpu.CompilerParams(needs_layout_passes=False),
)
def _sc_embed_add(idx_hbm, en_hbm, tab_hbm, out_hbm, idx_v, en_v, out_v, tab_v,
                  sem):
    wid = lax.axis_index("s") * _NC + lax.axis_index("c")
    base = lax.min(wid * _B_PER_W, _LAST_BASE)
    c1 = pltpu.async_copy(tab_hbm, tab_v, sem)
    c2 = pltpu.async_copy(idx_hbm.at[pl.ds(base, _B_PER_W)], idx_v, sem)
    c3 = pltpu.async_copy(en_hbm.at[pl.ds(base, _B_PER_W)], en_v, sem)
    c1.wait()
    c2.wait()
    c3.wait()

    @plsc.parallel_loop(0, _VECS, 1, unroll=_UNROLL)
    def body(j):
        off = j * _L
        iv = idx_v[pl.ds(off, _L)]
        g = plsc.load_gather(tab_v, [iv])
        out_v[pl.ds(off, _L)] = g + en_v[pl.ds(off, _L)]

    pltpu.sync_copy(out_v, out_hbm.at[pl.ds(base, _B_PER_W)])


def kernel(atom_type, pos, atomic_energy, emb_weight):
    out = _sc_embed_add(
        atom_type.reshape(-1),
        atomic_energy.reshape(-1),
        emb_weight.reshape(-1),
    )
    return out.reshape(_N, 1).astype(pos.dtype)
